# baseline (speedup 1.0000x reference)
import functools

import jax
import jax.numpy as jnp
from jax import lax
from jax.experimental import pallas as pl
from jax.experimental.pallas import tpu as pltpu
from jax.experimental.pallas import tpu_sc as plsc

D_MODEL = 256
K = 8
Q = 1024
N = 100000

BQ = 128
CN = 8192
NCH = 13
N_PAD = NCH * CN

_HUGE_F = 1e30
_HUGE_I = 2 ** 30


def _make_topk_body(bq, cn, nch):
    def _topk_body(qp_ref, sp_ref, qn_ref, pn_ref, out_ref, topv, topi, d2s,
                   mrow, cont):
        n = pl.program_id(1)

        @pl.when(n == 0)
        def _init():
            topv[:, :] = jnp.full((bq, K), _HUGE_F, jnp.float32)
            topi[:, :] = _HUGE_I + lax.broadcasted_iota(jnp.int32, (bq, K), 1)

        qp = jnp.dot(qp_ref[:, :], sp_ref[:, :],
                     preferred_element_type=jnp.float32)
        d2 = (qn_ref[:, :] + pn_ref[:, :]) - 2.0 * qp
        d2s[:, :] = d2
        m0 = jnp.min(d2, axis=1, keepdims=True)
        mrow[:, :] = m0
        cm0 = jnp.max(topv[:, :], axis=1, keepdims=True)
        cont[0] = jnp.max((m0 <= cm0).astype(jnp.int32))

        lane = lax.broadcasted_iota(jnp.int32, (bq, cn), 1)
        base = n * cn
        for _ in range(K):
            @pl.when(cont[0] != 0)
            def _extract():
                m = mrow[:, :]
                d2v = d2s[:, :]
                cand = jnp.where(d2v <= m, lane, _HUGE_I)
                am = jnp.min(cand, axis=1, keepdims=True)
                gi = base + am
                tv = topv[:, :]
                ti = topi[:, :]
                cm = jnp.max(tv, axis=1, keepdims=True)
                is_max = tv == cm
                ev_i = jnp.max(jnp.where(is_max, ti, -1), axis=1,
                               keepdims=True)
                ev_mask = is_max & (ti == ev_i)
                take = (m < cm) | ((m == cm) & (gi < ev_i))
                upd = ev_mask & take
                tv = jnp.where(upd, m, tv)
                ti = jnp.where(upd, gi, ti)
                topv[:, :] = tv
                topi[:, :] = ti
                d2m = jnp.where(lane == am, _HUGE_F, d2v)
                d2s[:, :] = d2m
                m2 = jnp.min(d2m, axis=1, keepdims=True)
                mrow[:, :] = m2
                cm2 = jnp.max(tv, axis=1, keepdims=True)
                cont[0] = jnp.max((m2 <= cm2).astype(jnp.int32))

        @pl.when(n == nch - 1)
        def _emit():
            sv = topv[:, :]
            si = topi[:, :]
            cols = []
            for _ in range(K):
                m = jnp.min(sv, axis=1, keepdims=True)
                am = jnp.min(jnp.where(sv <= m, si, _HUGE_I), axis=1,
                             keepdims=True)
                cols.append(am)
                sv = jnp.where((sv <= m) & (si == am), _HUGE_F, sv)
            out_ref[:, :] = jnp.concatenate(cols, axis=1)

    return _topk_body


def _make_topk(q, bq, cn, nch, semantics=("parallel", "arbitrary")):
    return pl.pallas_call(
        _make_topk_body(bq, cn, nch),
        grid=(q // bq, nch),
        in_specs=[
            pl.BlockSpec((bq, 3), lambda i, n: (i, 0)),
            pl.BlockSpec((3, cn), lambda i, n: (0, n)),
            pl.BlockSpec((bq, 1), lambda i, n: (i, 0)),
            pl.BlockSpec((1, cn), lambda i, n: (0, n)),
        ],
        out_specs=pl.BlockSpec((bq, K), lambda i, n: (i, 0)),
        out_shape=jax.ShapeDtypeStruct((q, K), jnp.int32),
        scratch_shapes=[
            pltpu.VMEM((bq, K), jnp.float32),
            pltpu.VMEM((bq, K), jnp.int32),
            pltpu.VMEM((bq, cn), jnp.float32),
            pltpu.VMEM((bq, 1), jnp.float32),
            pltpu.SMEM((1,), jnp.int32),
        ],
        compiler_params=pltpu.CompilerParams(
            dimension_semantics=semantics,
        ),
    )


def _topk_indices(query_pos, sp_t, qn, pn):
    return _make_topk(Q, BQ, CN, NCH)(query_pos, sp_t, qn, pn)


def _sc_gather(table, idx):
    info = plsc.get_sparse_core_info()
    nc, ns = info.num_cores, info.num_subcores
    nw = nc * ns
    b = idx.shape[0]
    b_per_w = b // nw
    d = table.shape[1]
    mesh = plsc.VectorSubcoreMesh(core_axis_name="c", subcore_axis_name="s")

    @functools.partial(
        pl.kernel,
        mesh=mesh,
        out_type=jax.ShapeDtypeStruct((b, d), jnp.float32),
        scratch_types=[
            pltpu.VMEM((b_per_w,), jnp.int32),
            pltpu.VMEM((b_per_w, d), jnp.float32),
            pltpu.SemaphoreType.DMA,
        ],
    )
    def gk(table_hbm, idx_hbm, out_hbm, idx_v, rows_v, sem):
        wid = lax.axis_index("s") * nc + lax.axis_index("c")
        base = wid * b_per_w
        pltpu.sync_copy(idx_hbm.at[pl.ds(base, b_per_w)], idx_v)
        pltpu.async_copy(table_hbm.at[idx_v], rows_v, sem).wait()
        pltpu.sync_copy(rows_v, out_hbm.at[pl.ds(base, b_per_w)])

    return gk(table, idx)


def _tail_body(query_ref, feat_ref, wq_t_ref, wv_t_ref, wo_t_ref, wk_t_ref,
               wb_ref, g_ref, b_ref, out_ref):
    query = query_ref[:, :]
    q = jnp.dot(query, wq_t_ref[:, :], preferred_element_type=jnp.float32)
    v = jnp.dot(feat_ref[:, :], wv_t_ref[:, :],
                preferred_element_type=jnp.float32)
    kl = jnp.dot(q, wk_t_ref[:, :],
                 preferred_element_type=jnp.float32) + wb_ref[:, :]
    kl = kl - jnp.max(kl, axis=1, keepdims=True)
    e = jnp.exp(kl)
    ks = e / jnp.sum(e, axis=1, keepdims=True)
    v3 = v.reshape(Q, K, D_MODEL)
    vs = jnp.sum(v3 * ks[:, :, None], axis=1)
    w = q * vs
    out = jnp.dot(w, wo_t_ref[:, :],
                  preferred_element_type=jnp.float32) + query
    mu = jnp.mean(out, axis=1, keepdims=True)
    var = jnp.mean((out - mu) ** 2, axis=1, keepdims=True)
    out_ref[:, :] = (out - mu) / jnp.sqrt(var + 1e-5) * g_ref[:, :] + b_ref[:, :]


def _tail(query, feat, w_q, w_v, w_o, w_k, w_b, ln_g, ln_b):
    return pl.pallas_call(
        _tail_body,
        out_shape=jax.ShapeDtypeStruct((Q, D_MODEL), jnp.float32),
    )(query, feat, w_q.T, w_v.T, w_o.T, w_k.T,
      w_b.reshape(1, K), ln_g.reshape(1, D_MODEL), ln_b.reshape(1, D_MODEL))


def kernel(query, query_pos, inst_feats, sp_coords, w_q, w_v, w_o, w_k, w_b,
           ln_g, ln_b):
    sp_pad = jnp.pad(sp_coords, ((0, N_PAD - N), (0, 0)),
                     constant_values=1e4)
    sp_t = sp_pad.T
    qn = jnp.sum(query_pos * query_pos, axis=-1, keepdims=True)
    pn = jnp.sum(sp_pad * sp_pad, axis=-1).reshape(1, N_PAD)
    idx = _topk_indices(query_pos, sp_t, qn, pn)
    feat = _sc_gather(inst_feats, idx.reshape(Q * K))
    return _tail(query, feat, w_q, w_v, w_o, w_k, w_b, ln_g, ln_b)


# speedup vs baseline: 1.0996x; 1.0996x over previous
import functools

import jax
import jax.numpy as jnp
from jax import lax
from jax.experimental import pallas as pl
from jax.experimental.pallas import tpu as pltpu
from jax.experimental.pallas import tpu_sc as plsc

D_MODEL = 256
K = 8
Q = 1024
N = 100000

BQ = 256
CN = 4096
NCH = 25
N_PAD = NCH * CN

_HUGE_F = 1e30
_HUGE_I = 2 ** 30


def _make_topk_body(bq, cn, nch):
    def _topk_body(qp_ref, sp_ref, qn_ref, pn_ref, out_ref, topv, topi, d2s,
                   mrow, cont):
        n = pl.program_id(1)

        @pl.when(n == 0)
        def _init():
            topv[:, :] = jnp.full((bq, K), _HUGE_F, jnp.float32)
            topi[:, :] = _HUGE_I + lax.broadcasted_iota(jnp.int32, (bq, K), 1)

        qp = jnp.dot(qp_ref[:, :], sp_ref[:, :],
                     preferred_element_type=jnp.float32)
        d2 = (qn_ref[:, :] + pn_ref[:, :]) - 2.0 * qp
        d2s[:, :] = d2
        m0 = jnp.min(d2, axis=1, keepdims=True)
        mrow[:, :] = m0
        cm0 = jnp.max(topv[:, :], axis=1, keepdims=True)
        cont[0] = jnp.max((m0 <= cm0).astype(jnp.int32))

        lane = lax.broadcasted_iota(jnp.int32, (bq, cn), 1)
        base = n * cn
        for _ in range(K):
            @pl.when(cont[0] != 0)
            def _extract():
                m = mrow[:, :]
                d2v = d2s[:, :]
                cand = jnp.where(d2v <= m, lane, _HUGE_I)
                am = jnp.min(cand, axis=1, keepdims=True)
                gi = base + am
                tv = topv[:, :]
                ti = topi[:, :]
                cm = jnp.max(tv, axis=1, keepdims=True)
                is_max = tv == cm
                ev_i = jnp.max(jnp.where(is_max, ti, -1), axis=1,
                               keepdims=True)
                ev_mask = is_max & (ti == ev_i)
                take = (m < cm) | ((m == cm) & (gi < ev_i))
                upd = ev_mask & take
                tv = jnp.where(upd, m, tv)
                ti = jnp.where(upd, gi, ti)
                topv[:, :] = tv
                topi[:, :] = ti
                d2m = jnp.where(lane == am, _HUGE_F, d2v)
                d2s[:, :] = d2m
                m2 = jnp.min(d2m, axis=1, keepdims=True)
                mrow[:, :] = m2
                cm2 = jnp.max(tv, axis=1, keepdims=True)
                cont[0] = jnp.max((m2 <= cm2).astype(jnp.int32))

        @pl.when(n == nch - 1)
        def _emit():
            sv = topv[:, :]
            si = topi[:, :]
            cols = []
            for _ in range(K):
                m = jnp.min(sv, axis=1, keepdims=True)
                am = jnp.min(jnp.where(sv <= m, si, _HUGE_I), axis=1,
                             keepdims=True)
                cols.append(am)
                sv = jnp.where((sv <= m) & (si == am), _HUGE_F, sv)
            out_ref[:, :] = jnp.concatenate(cols, axis=1)

    return _topk_body


def _make_topk(q, bq, cn, nch, semantics=("parallel", "arbitrary")):
    return pl.pallas_call(
        _make_topk_body(bq, cn, nch),
        grid=(q // bq, nch),
        in_specs=[
            pl.BlockSpec((bq, 3), lambda i, n: (i, 0)),
            pl.BlockSpec((3, cn), lambda i, n: (0, n)),
            pl.BlockSpec((bq, 1), lambda i, n: (i, 0)),
            pl.BlockSpec((1, cn), lambda i, n: (0, n)),
        ],
        out_specs=pl.BlockSpec((bq, K), lambda i, n: (i, 0)),
        out_shape=jax.ShapeDtypeStruct((q, K), jnp.int32),
        scratch_shapes=[
            pltpu.VMEM((bq, K), jnp.float32),
            pltpu.VMEM((bq, K), jnp.int32),
            pltpu.VMEM((bq, cn), jnp.float32),
            pltpu.VMEM((bq, 1), jnp.float32),
            pltpu.SMEM((1,), jnp.int32),
        ],
        compiler_params=pltpu.CompilerParams(
            dimension_semantics=semantics,
        ),
    )


def _topk_indices(query_pos, sp_t, qn, pn):
    return _make_topk(Q, BQ, CN, NCH)(query_pos, sp_t, qn, pn)


def _sc_gather(table, idx):
    info = plsc.get_sparse_core_info()
    nc, ns = info.num_cores, info.num_subcores
    nw = nc * ns
    b = idx.shape[0]
    b_per_w = b // nw
    d = table.shape[1]
    mesh = plsc.VectorSubcoreMesh(core_axis_name="c", subcore_axis_name="s")

    @functools.partial(
        pl.kernel,
        mesh=mesh,
        out_type=jax.ShapeDtypeStruct((b, d), jnp.float32),
        scratch_types=[
            pltpu.VMEM((b_per_w,), jnp.int32),
            pltpu.VMEM((b_per_w, d), jnp.float32),
            pltpu.SemaphoreType.DMA,
        ],
    )
    def gk(table_hbm, idx_hbm, out_hbm, idx_v, rows_v, sem):
        wid = lax.axis_index("s") * nc + lax.axis_index("c")
        base = wid * b_per_w
        pltpu.sync_copy(idx_hbm.at[pl.ds(base, b_per_w)], idx_v)
        pltpu.async_copy(table_hbm.at[idx_v], rows_v, sem).wait()
        pltpu.sync_copy(rows_v, out_hbm.at[pl.ds(base, b_per_w)])

    return gk(table, idx)


def _tail_body(query_ref, feat_ref, wq_t_ref, wv_t_ref, wo_t_ref, wk_t_ref,
               wb_ref, g_ref, b_ref, out_ref):
    query = query_ref[:, :]
    q = jnp.dot(query, wq_t_ref[:, :], preferred_element_type=jnp.float32)
    v = jnp.dot(feat_ref[:, :], wv_t_ref[:, :],
                preferred_element_type=jnp.float32)
    kl = jnp.dot(q, wk_t_ref[:, :],
                 preferred_element_type=jnp.float32) + wb_ref[:, :]
    kl = kl - jnp.max(kl, axis=1, keepdims=True)
    e = jnp.exp(kl)
    ks = e / jnp.sum(e, axis=1, keepdims=True)
    v3 = v.reshape(Q, K, D_MODEL)
    vs = jnp.sum(v3 * ks[:, :, None], axis=1)
    w = q * vs
    out = jnp.dot(w, wo_t_ref[:, :],
                  preferred_element_type=jnp.float32) + query
    mu = jnp.mean(out, axis=1, keepdims=True)
    var = jnp.mean((out - mu) ** 2, axis=1, keepdims=True)
    out_ref[:, :] = (out - mu) / jnp.sqrt(var + 1e-5) * g_ref[:, :] + b_ref[:, :]


def _tail(query, feat, w_q, w_v, w_o, w_k, w_b, ln_g, ln_b):
    return pl.pallas_call(
        _tail_body,
        out_shape=jax.ShapeDtypeStruct((Q, D_MODEL), jnp.float32),
    )(query, feat, w_q.T, w_v.T, w_o.T, w_k.T,
      w_b.reshape(1, K), ln_g.reshape(1, D_MODEL), ln_b.reshape(1, D_MODEL))


def kernel(query, query_pos, inst_feats, sp_coords, w_q, w_v, w_o, w_k, w_b,
           ln_g, ln_b):
    sp_pad = jnp.pad(sp_coords, ((0, N_PAD - N), (0, 0)),
                     constant_values=1e4)
    sp_t = sp_pad.T
    qn = jnp.sum(query_pos * query_pos, axis=-1, keepdims=True)
    pn = jnp.sum(sp_pad * sp_pad, axis=-1).reshape(1, N_PAD)
    idx = _topk_indices(query_pos, sp_t, qn, pn)
    feat = _sc_gather(inst_feats, idx.reshape(Q * K))
    return _tail(query, feat, w_q, w_v, w_o, w_k, w_b, ln_g, ln_b)


# speedup vs baseline: 1.2242x; 1.1133x over previous
import functools

import jax
import jax.numpy as jnp
from jax import lax
from jax.experimental import pallas as pl
from jax.experimental.pallas import tpu as pltpu
from jax.experimental.pallas import tpu_sc as plsc

D_MODEL = 256
K = 8
Q = 1024
N = 100000

BQ = 512
CN = 4096
NCH = 25
N_PAD = NCH * CN

_HUGE_F = 1e30
_HUGE_I = 2 ** 30


def _make_topk_body(bq, cn, nch):
    def _topk_body(qp_ref, sp_ref, qn_ref, pn_ref, out_ref, topv, topi, d2s,
                   mrow, cont):
        n = pl.program_id(1)

        @pl.when(n == 0)
        def _init():
            topv[:, :] = jnp.full((bq, K), _HUGE_F, jnp.float32)
            topi[:, :] = _HUGE_I + lax.broadcasted_iota(jnp.int32, (bq, K), 1)

        qp = jnp.dot(qp_ref[:, :], sp_ref[:, :],
                     preferred_element_type=jnp.float32)
        d2 = (qn_ref[:, :] + pn_ref[:, :]) - 2.0 * qp
        d2s[:, :] = d2
        m0 = jnp.min(d2, axis=1, keepdims=True)
        mrow[:, :] = m0
        cm0 = jnp.max(topv[:, :], axis=1, keepdims=True)
        cont[0] = jnp.max((m0 <= cm0).astype(jnp.int32))

        lane = lax.broadcasted_iota(jnp.int32, (bq, cn), 1)
        base = n * cn
        for _ in range(K):
            @pl.when(cont[0] != 0)
            def _extract():
                m = mrow[:, :]
                d2v = d2s[:, :]
                cand = jnp.where(d2v <= m, lane, _HUGE_I)
                am = jnp.min(cand, axis=1, keepdims=True)
                gi = base + am
                tv = topv[:, :]
                ti = topi[:, :]
                cm = jnp.max(tv, axis=1, keepdims=True)
                is_max = tv == cm
                ev_i = jnp.max(jnp.where(is_max, ti, -1), axis=1,
                               keepdims=True)
                ev_mask = is_max & (ti == ev_i)
                take = (m < cm) | ((m == cm) & (gi < ev_i))
                upd = ev_mask & take
                tv = jnp.where(upd, m, tv)
                ti = jnp.where(upd, gi, ti)
                topv[:, :] = tv
                topi[:, :] = ti
                d2m = jnp.where(lane == am, _HUGE_F, d2v)
                d2s[:, :] = d2m
                m2 = jnp.min(d2m, axis=1, keepdims=True)
                mrow[:, :] = m2
                cm2 = jnp.max(tv, axis=1, keepdims=True)
                cont[0] = jnp.max((m2 <= cm2).astype(jnp.int32))

        @pl.when(n == nch - 1)
        def _emit():
            sv = topv[:, :]
            si = topi[:, :]
            cols = []
            for _ in range(K):
                m = jnp.min(sv, axis=1, keepdims=True)
                am = jnp.min(jnp.where(sv <= m, si, _HUGE_I), axis=1,
                             keepdims=True)
                cols.append(am)
                sv = jnp.where((sv <= m) & (si == am), _HUGE_F, sv)
            out_ref[:, :] = jnp.concatenate(cols, axis=1)

    return _topk_body


def _make_topk(q, bq, cn, nch, semantics=("parallel", "arbitrary")):
    return pl.pallas_call(
        _make_topk_body(bq, cn, nch),
        grid=(q // bq, nch),
        in_specs=[
            pl.BlockSpec((bq, 3), lambda i, n: (i, 0)),
            pl.BlockSpec((3, cn), lambda i, n: (0, n)),
            pl.BlockSpec((bq, 1), lambda i, n: (i, 0)),
            pl.BlockSpec((1, cn), lambda i, n: (0, n)),
        ],
        out_specs=pl.BlockSpec((bq, K), lambda i, n: (i, 0)),
        out_shape=jax.ShapeDtypeStruct((q, K), jnp.int32),
        scratch_shapes=[
            pltpu.VMEM((bq, K), jnp.float32),
            pltpu.VMEM((bq, K), jnp.int32),
            pltpu.VMEM((bq, cn), jnp.float32),
            pltpu.VMEM((bq, 1), jnp.float32),
            pltpu.SMEM((1,), jnp.int32),
        ],
        compiler_params=pltpu.CompilerParams(
            dimension_semantics=semantics,
        ),
    )


def _topk_indices(query_pos, sp_t, qn, pn):
    return _make_topk(Q, BQ, CN, NCH)(query_pos, sp_t, qn, pn)


def _sc_gather(table, idx):
    info = plsc.get_sparse_core_info()
    nc, ns = info.num_cores, info.num_subcores
    nw = nc * ns
    b = idx.shape[0]
    b_per_w = b // nw
    d = table.shape[1]
    mesh = plsc.VectorSubcoreMesh(core_axis_name="c", subcore_axis_name="s")

    @functools.partial(
        pl.kernel,
        mesh=mesh,
        out_type=jax.ShapeDtypeStruct((b, d), jnp.float32),
        scratch_types=[
            pltpu.VMEM((b_per_w,), jnp.int32),
            pltpu.VMEM((b_per_w, d), jnp.float32),
            pltpu.SemaphoreType.DMA,
        ],
    )
    def gk(table_hbm, idx_hbm, out_hbm, idx_v, rows_v, sem):
        wid = lax.axis_index("s") * nc + lax.axis_index("c")
        base = wid * b_per_w
        pltpu.sync_copy(idx_hbm.at[pl.ds(base, b_per_w)], idx_v)
        pltpu.async_copy(table_hbm.at[idx_v], rows_v, sem).wait()
        pltpu.sync_copy(rows_v, out_hbm.at[pl.ds(base, b_per_w)])

    return gk(table, idx)


def _tail_body(query_ref, feat_ref, wq_t_ref, wv_t_ref, wo_t_ref, wk_t_ref,
               wb_ref, g_ref, b_ref, out_ref):
    query = query_ref[:, :]
    q = jnp.dot(query, wq_t_ref[:, :], preferred_element_type=jnp.float32)
    v = jnp.dot(feat_ref[:, :], wv_t_ref[:, :],
                preferred_element_type=jnp.float32)
    kl = jnp.dot(q, wk_t_ref[:, :],
                 preferred_element_type=jnp.float32) + wb_ref[:, :]
    kl = kl - jnp.max(kl, axis=1, keepdims=True)
    e = jnp.exp(kl)
    ks = e / jnp.sum(e, axis=1, keepdims=True)
    v3 = v.reshape(Q, K, D_MODEL)
    vs = jnp.sum(v3 * ks[:, :, None], axis=1)
    w = q * vs
    out = jnp.dot(w, wo_t_ref[:, :],
                  preferred_element_type=jnp.float32) + query
    mu = jnp.mean(out, axis=1, keepdims=True)
    var = jnp.mean((out - mu) ** 2, axis=1, keepdims=True)
    out_ref[:, :] = (out - mu) / jnp.sqrt(var + 1e-5) * g_ref[:, :] + b_ref[:, :]


def _tail(query, feat, w_q, w_v, w_o, w_k, w_b, ln_g, ln_b):
    return pl.pallas_call(
        _tail_body,
        out_shape=jax.ShapeDtypeStruct((Q, D_MODEL), jnp.float32),
    )(query, feat, w_q.T, w_v.T, w_o.T, w_k.T,
      w_b.reshape(1, K), ln_g.reshape(1, D_MODEL), ln_b.reshape(1, D_MODEL))


def kernel(query, query_pos, inst_feats, sp_coords, w_q, w_v, w_o, w_k, w_b,
           ln_g, ln_b):
    sp_pad = jnp.pad(sp_coords, ((0, N_PAD - N), (0, 0)),
                     constant_values=1e4)
    sp_t = sp_pad.T
    qn = jnp.sum(query_pos * query_pos, axis=-1, keepdims=True)
    pn = jnp.sum(sp_pad * sp_pad, axis=-1).reshape(1, N_PAD)
    idx = _topk_indices(query_pos, sp_t, qn, pn)
    feat = _sc_gather(inst_feats, idx.reshape(Q * K))
    return _tail(query, feat, w_q, w_v, w_o, w_k, w_b, ln_g, ln_b)
